# baseline (speedup 1.0000x reference)
import functools

import numpy as np

import jax
import jax.numpy as jnp
from jax import lax
from jax.experimental import pallas as pl
from jax.experimental.pallas import tpu as pltpu
from jax.experimental.pallas import tpu_sc as plsc

_COLS = 8
_EMB = 12
_D = 16
_B = 16384
_NPAIR = _COLS * (_COLS - 1) // 2
_F = _COLS * _D

_SROWS, _SCOLS, _SK, _SD = [], [], [], []
_k = 0
for _i in range(_COLS):
    for _j in range(_i + 1, _COLS):
        for _d in range(_D):
            _SROWS += [_i * _D + _d, _j * _D + _d]
            _SCOLS += [_j * _D + _d, _i * _D + _d]
            _SK += [_k, _k]
            _SD += [_d, _d]
        _k += 1
_SROWS = np.asarray(_SROWS, np.int32)
_SCOLS = np.asarray(_SCOLS, np.int32)
_SK = np.asarray(_SK, np.int32)
_SD = np.asarray(_SD, np.int32)


def _softplus_prep(sd_ref, out_ref):
    out_ref[...] = 0.01 * jnp.log(1.0 + jnp.exp(sd_ref[...]))


def _tc_pair(e_ref, s_ref, a_ref, out_ref):
    e = e_ref[...]
    p = jnp.dot(e, s_ref[...], preferred_element_type=jnp.float32)
    rs = jnp.sum(e * p, axis=1)
    ea = jnp.dot(e, a_ref[...], preferred_element_type=jnp.float32)
    out_ref[...] = 0.5 * rs[:, None] + ea


def kernel(workclass, education, marital_status, occupation, relationship,
           race, sex, native_country, mean_tables, std_tables, fc_weights,
           action_emb, rand_array):
    cols = [workclass, education, marital_status, occupation, relationship,
            race, sex, native_country]

    mu96 = mean_tables.reshape(_COLS * _EMB * _D)
    s96 = pl.pallas_call(
        _softplus_prep,
        out_shape=jax.ShapeDtypeStruct((_COLS * _EMB, _D), jnp.float32),
    )(std_tables.reshape(_COLS * _EMB, _D)).reshape(_COLS * _EMB * _D)

    smat = jnp.zeros((_F, _F), jnp.float32).at[_SROWS, _SCOLS].set(
        fc_weights[_SK, _SD])
    amat = jnp.stack([jnp.tile(action_emb[0], _COLS),
                      jnp.tile(action_emb[1], _COLS)], axis=1)

    info = plsc.get_sparse_core_info()
    nc = info.num_cores
    nw = info.num_cores * info.num_subcores
    rpw = _B // nw
    ngrp = rpw // _D

    mesh = plsc.VectorSubcoreMesh(core_axis_name="c", subcore_axis_name="s")

    @functools.partial(
        pl.kernel,
        out_type=jax.ShapeDtypeStruct((_B, _F), jnp.float32),
        mesh=mesh,
        compiler_params=pltpu.CompilerParams(needs_layout_passes=False),
        scratch_types=[
            pltpu.VMEM((_COLS, rpw), jnp.int32),
            pltpu.VMEM((rpw * _D,), jnp.float32),
            pltpu.VMEM((_COLS * _EMB * _D,), jnp.float32),
            pltpu.VMEM((_COLS * _EMB * _D,), jnp.float32),
            pltpu.VMEM((rpw, _F), jnp.float32),
            pltpu.SemaphoreType.DMA,
            pltpu.SemaphoreType.DMA,
        ],
    )
    def _fm_sc(c0, c1, c2, c3, c4, c5, c6, c7, mu_ref, s_ref, rnd_ref,
               e_out, idx_b, v_b, mu_v, s_v, e_b, sem, sem2):
        crefs = [c0, c1, c2, c3, c4, c5, c6, c7]
        wid = lax.axis_index("s") * nc + lax.axis_index("c")
        base = wid * rpw

        descs = [
            pltpu.async_copy(mu_ref, mu_v, sem),
            pltpu.async_copy(s_ref, s_v, sem),
            pltpu.async_copy(rnd_ref.at[pl.ds(base * _D, rpw * _D)], v_b, sem),
        ]
        descs += [
            pltpu.async_copy(crefs[i].at[pl.ds(base, rpw)], idx_b.at[i], sem)
            for i in range(_COLS)
        ]
        for d in descs:
            d.wait()

        for i in range(_COLS):
            off = jnp.full((_D,), _EMB * _D * i, jnp.int32)
            for t in range(rpw // _D):
                sl = pl.ds(t * _D, _D)
                idx_b[i, sl] = (idx_b[i, sl] * _D) + off

        lanes = jnp.arange(_D, dtype=jnp.int32)

        @plsc.parallel_loop(0, ngrp)
        def _grp(g):
            g16 = g * _D
            ivecs = [idx_b[i, pl.ds(g16, _D)] for i in range(_COLS)]
            for rl in range(_D):
                rlvec = jnp.full((_D,), rl, jnp.int32)
                v = v_b[pl.ds((g16 + rl) * _D, _D)]
                for i in range(_COLS):
                    addr = jnp.take_along_axis(ivecs[i], rlvec, axis=0) + lanes
                    mu_i = plsc.load_gather(mu_v, [addr])
                    s_i = plsc.load_gather(s_v, [addr])
                    e_b[g16 + rl, pl.ds(i * _D, _D)] = mu_i + s_i * v
            pltpu.async_copy(e_b.at[pl.ds(g16, _D)],
                             e_out.at[pl.ds(base + g16, _D)], sem2)

        for _ in range(ngrp):
            pltpu.make_async_copy(
                e_out.at[pl.ds(base, _D)], e_b.at[pl.ds(0, _D)], sem2).wait()

    e_mat = _fm_sc(*cols, mu96, s96, rand_array)

    tile = 512
    out = pl.pallas_call(
        _tc_pair,
        out_shape=jax.ShapeDtypeStruct((_B, 2), jnp.float32),
        grid=(_B // tile,),
        in_specs=[
            pl.BlockSpec((tile, _F), lambda i: (i, 0)),
            pl.BlockSpec((_F, _F), lambda i: (0, 0)),
            pl.BlockSpec((_F, 2), lambda i: (0, 0)),
        ],
        out_specs=pl.BlockSpec((tile, 2), lambda i: (i, 0)),
    )(e_mat, smat, amat)
    return out


# speedup vs baseline: 1.3324x; 1.3324x over previous
import functools

import jax
import jax.numpy as jnp
from jax import lax
from jax.experimental import pallas as pl
from jax.experimental.pallas import tpu as pltpu
from jax.experimental.pallas import tpu_sc as plsc

_COLS = 8
_EMB = 12
_D = 16
_B = 16384
_NPAIR = _COLS * (_COLS - 1) // 2
_PIN = _NPAIR


def _tsum(xs):
    xs = list(xs)
    while len(xs) > 1:
        nxt = [a + b for a, b in zip(xs[::2], xs[1::2])]
        if len(xs) % 2:
            nxt.append(xs[-1])
        xs = nxt
    return xs[0]


def _softplus_prep(sd_ref, out_ref):
    out_ref[...] = 0.01 * jnp.log(1.0 + jnp.exp(sd_ref[...]))


def kernel(workclass, education, marital_status, occupation, relationship,
           race, sex, native_country, mean_tables, std_tables, fc_weights,
           action_emb, rand_array):
    cols = [workclass, education, marital_status, occupation, relationship,
            race, sex, native_country]

    mu96 = mean_tables.reshape(_COLS * _EMB * _D)
    s96 = pl.pallas_call(
        _softplus_prep,
        out_shape=jax.ShapeDtypeStruct((_COLS * _EMB, _D), jnp.float32),
    )(std_tables.reshape(_COLS * _EMB, _D)).reshape(_COLS * _EMB * _D)

    info = plsc.get_sparse_core_info()
    nc = info.num_cores
    nw = info.num_cores * info.num_subcores
    rpw = _B // nw
    ngrp = rpw // _D

    mesh = plsc.VectorSubcoreMesh(core_axis_name="c", subcore_axis_name="s")

    @functools.partial(
        pl.kernel,
        out_type=jax.ShapeDtypeStruct((_B, 2), jnp.float32),
        mesh=mesh,
        compiler_params=pltpu.CompilerParams(needs_layout_passes=False),
        scratch_types=[
            pltpu.VMEM((_COLS, rpw), jnp.int32),
            pltpu.VMEM((rpw * _D,), jnp.float32),
            pltpu.VMEM((_COLS * _EMB * _D,), jnp.float32),
            pltpu.VMEM((_COLS * _EMB * _D,), jnp.float32),
            pltpu.VMEM((_NPAIR, _D), jnp.float32),
            pltpu.VMEM((2, _D), jnp.float32),
            pltpu.VMEM((rpw, 2), jnp.float32),
            pltpu.SemaphoreType.DMA,
        ],
    )
    def _fm_sc(c0, c1, c2, c3, c4, c5, c6, c7, mu_ref, s_ref, w_ref, a_ref,
               rnd_ref, out_ref, idx_b, v_b, mu_v, s_v, w_b, a_b, o_b, sem):
        crefs = [c0, c1, c2, c3, c4, c5, c6, c7]
        wid = lax.axis_index("s") * nc + lax.axis_index("c")
        base = wid * rpw

        descs = [
            pltpu.async_copy(mu_ref, mu_v, sem),
            pltpu.async_copy(s_ref, s_v, sem),
            pltpu.async_copy(w_ref, w_b, sem),
            pltpu.async_copy(a_ref, a_b, sem),
            pltpu.async_copy(rnd_ref.at[pl.ds(base * _D, rpw * _D)], v_b, sem),
        ]
        descs += [
            pltpu.async_copy(crefs[i].at[pl.ds(base, rpw)], idx_b.at[i], sem)
            for i in range(_COLS)
        ]
        for d in descs:
            d.wait()

        for i in range(_COLS):
            off = jnp.full((_D,), _EMB * _D * i, jnp.int32)
            for t in range(rpw // _D):
                sl = pl.ds(t * _D, _D)
                idx_b[i, sl] = (idx_b[i, sl] * _D) + off

        w_pin = [w_b[k] for k in range(_PIN)]
        a0 = a_b[0]
        a1 = a_b[1]
        lanes = jnp.arange(_D, dtype=jnp.int32)
        m15 = lanes == (_D - 1)
        col0 = jnp.zeros((_D,), jnp.int32)
        col1 = jnp.full((_D,), 1, jnp.int32)

        @plsc.parallel_loop(0, ngrp, unroll=2)
        def _grp(g):
            g16 = g * _D
            ivecs = [idx_b[i, pl.ds(g16, _D)] for i in range(_COLS)]
            for rl in range(_D):
                rlvec = jnp.full((_D,), rl, jnp.int32)
                e = []
                for i in range(_COLS):
                    addr = jnp.take_along_axis(ivecs[i], rlvec, axis=0) + lanes
                    mu_i = plsc.load_gather(mu_v, [addr])
                    s_i = plsc.load_gather(s_v, [addr])
                    e.append((mu_i, s_i))
                v = v_b[pl.ds((g16 + rl) * _D, _D)]
                e = [mu_i + s_i * v for (mu_i, s_i) in e]
                k = 0
                acc = None
                for i in range(_COLS - 1):
                    gg = None
                    for j in range(i + 1, _COLS):
                        term = w_pin[k] * e[j]
                        gg = term if gg is None else gg + term
                        k += 1
                    t = e[i] * gg
                    acc = t if acc is None else acc + t
                se = e[0]
                for i in range(1, _COLS):
                    se = se + e[i]
                z0 = acc + se * a0
                z1 = acc + se * a1
                cz0 = plsc.cumsum(z0)
                cz1 = plsc.cumsum(z1)
                ridx = jnp.full((_D,), g16 + rl, jnp.int32)
                plsc.store_scatter(o_b, [ridx, col0], cz0, mask=m15)
                plsc.store_scatter(o_b, [ridx, col1], cz1, mask=m15)

        pltpu.sync_copy(o_b, out_ref.at[pl.ds(base, rpw)])

    return _fm_sc(*cols, mu96, s96, fc_weights, action_emb, rand_array)


# speedup vs baseline: 1.8838x; 1.4138x over previous
import functools

import jax
import jax.numpy as jnp
from jax import lax
from jax.experimental import pallas as pl
from jax.experimental.pallas import tpu as pltpu
from jax.experimental.pallas import tpu_sc as plsc

_COLS = 8
_EMB = 12
_D = 16
_B = 16384
_NPAIR = _COLS * (_COLS - 1) // 2
_PIN = _NPAIR


def _tsum(xs):
    xs = list(xs)
    while len(xs) > 1:
        nxt = [a + b for a, b in zip(xs[::2], xs[1::2])]
        if len(xs) % 2:
            nxt.append(xs[-1])
        xs = nxt
    return xs[0]


def _softplus_prep(sd_ref, out_ref):
    out_ref[...] = 0.01 * jnp.log(1.0 + jnp.exp(sd_ref[...]))


def kernel(workclass, education, marital_status, occupation, relationship,
           race, sex, native_country, mean_tables, std_tables, fc_weights,
           action_emb, rand_array):
    cols = [workclass, education, marital_status, occupation, relationship,
            race, sex, native_country]

    mu96 = mean_tables.reshape(_COLS * _EMB * _D)
    s96 = pl.pallas_call(
        _softplus_prep,
        out_shape=jax.ShapeDtypeStruct((_COLS * _EMB, _D), jnp.float32),
    )(std_tables.reshape(_COLS * _EMB, _D)).reshape(_COLS * _EMB * _D)

    info = plsc.get_sparse_core_info()
    nc = info.num_cores
    nw = info.num_cores * info.num_subcores
    rpw = _B // nw
    ngrp = rpw // _D

    mesh = plsc.VectorSubcoreMesh(core_axis_name="c", subcore_axis_name="s")

    @functools.partial(
        pl.kernel,
        out_type=jax.ShapeDtypeStruct((_B, 2), jnp.float32),
        mesh=mesh,
        compiler_params=pltpu.CompilerParams(needs_layout_passes=False),
        scratch_types=[
            pltpu.VMEM((_COLS, rpw), jnp.int32),
            pltpu.VMEM((rpw * _D,), jnp.float32),
            pltpu.VMEM((_COLS * _EMB * _D,), jnp.float32),
            pltpu.VMEM((_COLS * _EMB * _D,), jnp.float32),
            pltpu.VMEM((_NPAIR, _D), jnp.float32),
            pltpu.VMEM((2, _D), jnp.float32),
            pltpu.VMEM((rpw, 2), jnp.float32),
            pltpu.SemaphoreType.DMA,
        ],
    )
    def _fm_sc(c0, c1, c2, c3, c4, c5, c6, c7, mu_ref, s_ref, w_ref, a_ref,
               rnd_ref, out_ref, idx_b, v_b, mu_v, s_v, w_b, a_b, o_b, sem):
        crefs = [c0, c1, c2, c3, c4, c5, c6, c7]
        wid = lax.axis_index("s") * nc + lax.axis_index("c")
        base = wid * rpw

        descs = [
            pltpu.async_copy(mu_ref, mu_v, sem),
            pltpu.async_copy(s_ref, s_v, sem),
            pltpu.async_copy(w_ref, w_b, sem),
            pltpu.async_copy(a_ref, a_b, sem),
            pltpu.async_copy(rnd_ref.at[pl.ds(base * _D, rpw * _D)], v_b, sem),
        ]
        descs += [
            pltpu.async_copy(crefs[i].at[pl.ds(base, rpw)], idx_b.at[i], sem)
            for i in range(_COLS)
        ]
        for d in descs:
            d.wait()

        for i in range(_COLS):
            off = jnp.full((_D,), _EMB * _D * i, jnp.int32)
            for t in range(rpw // _D):
                sl = pl.ds(t * _D, _D)
                idx_b[i, sl] = (idx_b[i, sl] * _D) + off

        w_pin = [w_b[k] for k in range(_PIN)]
        a0 = a_b[0]
        a1 = a_b[1]
        lanes = jnp.arange(_D, dtype=jnp.int32)
        m15 = lanes == (_D - 1)
        col0 = jnp.zeros((_D,), jnp.int32)
        col1 = jnp.full((_D,), 1, jnp.int32)

        @plsc.parallel_loop(0, ngrp)
        def _grp(g):
            g16 = g * _D
            ivecs = [idx_b[i, pl.ds(g16, _D)] for i in range(_COLS)]
            for rl in range(_D):
                rlvec = jnp.full((_D,), rl, jnp.int32)
                e = []
                for i in range(_COLS):
                    addr = jnp.take_along_axis(ivecs[i], rlvec, axis=0) + lanes
                    mu_i = plsc.load_gather(mu_v, [addr])
                    s_i = plsc.load_gather(s_v, [addr])
                    e.append((mu_i, s_i))
                v = v_b[pl.ds((g16 + rl) * _D, _D)]
                e = [mu_i + s_i * v for (mu_i, s_i) in e]
                k = 0
                acc = None
                for i in range(_COLS - 1):
                    gg = None
                    for j in range(i + 1, _COLS):
                        term = w_pin[k] * e[j]
                        gg = term if gg is None else gg + term
                        k += 1
                    t = e[i] * gg
                    acc = t if acc is None else acc + t
                se = e[0]
                for i in range(1, _COLS):
                    se = se + e[i]
                z0 = acc + se * a0
                z1 = acc + se * a1
                cz0 = plsc.cumsum(z0)
                cz1 = plsc.cumsum(z1)
                ridx = jnp.full((_D,), g16 + rl, jnp.int32)
                plsc.store_scatter(o_b, [ridx, col0], cz0, mask=m15)
                plsc.store_scatter(o_b, [ridx, col1], cz1, mask=m15)

        pltpu.sync_copy(o_b, out_ref.at[pl.ds(base, rpw)])

    return _fm_sc(*cols, mu96, s96, fc_weights, action_emb, rand_array)


# speedup vs baseline: 1.8921x; 1.0044x over previous
import functools

import jax
import jax.numpy as jnp
from jax import lax
from jax.experimental import pallas as pl
from jax.experimental.pallas import tpu as pltpu
from jax.experimental.pallas import tpu_sc as plsc

_COLS = 8
_EMB = 12
_D = 16
_B = 16384
_NPAIR = _COLS * (_COLS - 1) // 2
def _softplus_prep(sd_ref, out_ref):
    out_ref[...] = 0.01 * jnp.log(1.0 + jnp.exp(sd_ref[...]))


def kernel(workclass, education, marital_status, occupation, relationship,
           race, sex, native_country, mean_tables, std_tables, fc_weights,
           action_emb, rand_array):
    cols = [workclass, education, marital_status, occupation, relationship,
            race, sex, native_country]

    mu96 = mean_tables.reshape(_COLS * _EMB * _D)
    s96 = pl.pallas_call(
        _softplus_prep,
        out_shape=jax.ShapeDtypeStruct((_COLS * _EMB, _D), jnp.float32),
    )(std_tables.reshape(_COLS * _EMB, _D)).reshape(_COLS * _EMB * _D)

    info = plsc.get_sparse_core_info()
    nc = info.num_cores
    nw = info.num_cores * info.num_subcores
    rpw = _B // nw
    ngrp = rpw // _D

    mesh = plsc.VectorSubcoreMesh(core_axis_name="c", subcore_axis_name="s")

    @functools.partial(
        pl.kernel,
        out_type=jax.ShapeDtypeStruct((_B, 2), jnp.float32),
        mesh=mesh,
        compiler_params=pltpu.CompilerParams(needs_layout_passes=False),
        scratch_types=[
            pltpu.VMEM((_COLS, rpw), jnp.int32),
            pltpu.VMEM((rpw * _D,), jnp.float32),
            pltpu.VMEM((_COLS * _EMB * _D,), jnp.float32),
            pltpu.VMEM((_COLS * _EMB * _D,), jnp.float32),
            pltpu.VMEM((_NPAIR, _D), jnp.float32),
            pltpu.VMEM((2, _D), jnp.float32),
            pltpu.VMEM((rpw, 2), jnp.float32),
            pltpu.SemaphoreType.DMA,
        ],
    )
    def _fm_sc(c0, c1, c2, c3, c4, c5, c6, c7, mu_ref, s_ref, w_ref, a_ref,
               rnd_ref, out_ref, idx_b, v_b, mu_v, s_v, w_b, a_b, o_b, sem):
        crefs = [c0, c1, c2, c3, c4, c5, c6, c7]
        wid = lax.axis_index("s") * nc + lax.axis_index("c")
        base = wid * rpw

        descs = [
            pltpu.async_copy(mu_ref, mu_v, sem),
            pltpu.async_copy(s_ref, s_v, sem),
            pltpu.async_copy(w_ref, w_b, sem),
            pltpu.async_copy(a_ref, a_b, sem),
            pltpu.async_copy(rnd_ref.at[pl.ds(base * _D, rpw * _D)], v_b, sem),
        ]
        descs += [
            pltpu.async_copy(crefs[i].at[pl.ds(base, rpw)], idx_b.at[i], sem)
            for i in range(_COLS)
        ]
        for d in descs:
            d.wait()

        for i in range(_COLS):
            off = jnp.full((_D,), _EMB * _D * i, jnp.int32)
            for t in range(rpw // _D):
                sl = pl.ds(t * _D, _D)
                idx_b[i, sl] = (idx_b[i, sl] * _D) + off

        w_pin = [w_b[k] for k in range(_NPAIR)]
        a0 = a_b[0]
        a1 = a_b[1]
        lanes = jnp.arange(_D, dtype=jnp.int32)
        m15 = lanes == (_D - 1)
        col0 = jnp.zeros((_D,), jnp.int32)
        col1 = jnp.full((_D,), 1, jnp.int32)

        @plsc.parallel_loop(0, ngrp * 2)
        def _grp(h):
            g16 = (h // 2) * _D
            half = (h % 2) * (_D // 2)
            ivecs = [idx_b[i, pl.ds(g16, _D)] for i in range(_COLS)]
            for rl0 in range(_D // 2):
                rl = half + rl0
                rlvec = jnp.full((_D,), rl, jnp.int32)
                e = []
                for i in range(_COLS):
                    addr = jnp.take_along_axis(ivecs[i], rlvec, axis=0) + lanes
                    mu_i = plsc.load_gather(mu_v, [addr])
                    s_i = plsc.load_gather(s_v, [addr])
                    e.append((mu_i, s_i))
                v = v_b[pl.ds((g16 + rl) * _D, _D)]
                e = [mu_i + s_i * v for (mu_i, s_i) in e]
                k = 0
                acc = None
                for i in range(_COLS - 1):
                    gg = None
                    for j in range(i + 1, _COLS):
                        term = w_pin[k] * e[j]
                        gg = term if gg is None else gg + term
                        k += 1
                    t = e[i] * gg
                    acc = t if acc is None else acc + t
                se = e[0]
                for i in range(1, _COLS):
                    se = se + e[i]
                z0 = acc + se * a0
                z1 = acc + se * a1
                cz0 = plsc.cumsum(z0)
                cz1 = plsc.cumsum(z1)
                ridx = jnp.full((_D,), g16 + rl, jnp.int32)
                plsc.store_scatter(o_b, [ridx, col0], cz0, mask=m15)
                plsc.store_scatter(o_b, [ridx, col1], cz1, mask=m15)

        pltpu.sync_copy(o_b, out_ref.at[pl.ds(base, rpw)])

    return _fm_sc(*cols, mu96, s96, fc_weights, action_emb, rand_array)
